# baseline (speedup 1.0000x reference)
import jax
import jax.numpy as jnp
from jax import lax
from jax.experimental import pallas as pl
from jax.experimental.pallas import tpu as pltpu
from jax.experimental.pallas import tpu_sc as plsc

_NUM_EMB = 1024
_DIM = 256
_BLK = 512
_COMMIT = 0.25
_NC = 2
_NS = 16
_CHUNK = 128


def _vq_tc_kernel(x_ref, e_ref, enc_ref, idx_ref, loss_ref):
    i = pl.program_id(0)
    x = x_ref[...]
    e = e_ref[...]
    term1 = jnp.sum(x * x, axis=1, keepdims=True)
    term2 = jnp.sum(e * e, axis=1)
    term3 = jnp.dot(x, e.T, preferred_element_type=jnp.float32)
    dist = (term1 + term2[None, :]) - 2.0 * term3
    min_d = jnp.min(dist, axis=1, keepdims=True)
    col = jax.lax.broadcasted_iota(jnp.int32, (_BLK, _NUM_EMB), 1)
    idx = jnp.min(jnp.where(dist == min_d, col, _NUM_EMB), axis=1)
    enc = (col == idx[:, None]).astype(jnp.float32)
    enc_ref[...] = enc
    idx_ref[...] = idx
    part = jnp.sum(min_d)

    @pl.when(i == 0)
    def _():
        loss_ref[0, 0] = 0.0

    loss_ref[0, 0] += part


def _sc_gather_body(table_ref, idx_ref, out_ref, idx_v, rows_v, sem):
    wid = lax.axis_index("c") * _NS + lax.axis_index("s")
    n_per_w = out_ref.shape[0] // (_NC * _NS)
    base = wid * n_per_w
    for c in range(n_per_w // _CHUNK):
        off = base + c * _CHUNK
        pltpu.sync_copy(idx_ref.at[pl.ds(off, _CHUNK)], idx_v)
        pltpu.async_copy(table_ref.at[idx_v], rows_v, sem).wait()
        pltpu.sync_copy(rows_v, out_ref.at[pl.ds(off, _CHUNK)])


def kernel(inputs, embedding):
    input_shape = inputs.shape
    flat = inputs.reshape(-1, _DIM)
    n = flat.shape[0]
    grid = (n // _BLK,)
    enc, idx, loss_sum = pl.pallas_call(
        _vq_tc_kernel,
        grid=grid,
        in_specs=[
            pl.BlockSpec((_BLK, _DIM), lambda i: (i, 0)),
            pl.BlockSpec((_NUM_EMB, _DIM), lambda i: (0, 0)),
        ],
        out_specs=[
            pl.BlockSpec((_BLK, _NUM_EMB), lambda i: (i, 0)),
            pl.BlockSpec((_BLK,), lambda i: (i,)),
            pl.BlockSpec((1, 1), lambda i: (0, 0), memory_space=pltpu.SMEM),
        ],
        out_shape=[
            jax.ShapeDtypeStruct((n, _NUM_EMB), jnp.float32),
            jax.ShapeDtypeStruct((n,), jnp.int32),
            jax.ShapeDtypeStruct((1, 1), jnp.float32),
        ],
    )(flat, embedding)
    q = pl.kernel(
        _sc_gather_body,
        out_type=jax.ShapeDtypeStruct((n, _DIM), jnp.float32),
        mesh=plsc.VectorSubcoreMesh(
            core_axis_name="c", subcore_axis_name="s",
            num_cores=_NC, num_subcores=_NS),
        scratch_types=[
            pltpu.VMEM((_CHUNK,), jnp.int32),
            pltpu.VMEM((_CHUNK, _DIM), jnp.float32),
            pltpu.SemaphoreType.DMA,
        ],
    )(embedding, idx)
    loss = loss_sum[0, 0] * ((1.0 + _COMMIT) / (n * _DIM))
    quantized = q.reshape(input_shape[0], -1)
    return (loss, quantized, enc)


# speedup vs baseline: 1.4503x; 1.4503x over previous
import jax
import jax.numpy as jnp
from jax.experimental import pallas as pl
from jax.experimental.pallas import tpu as pltpu

_NUM_EMB = 1024
_DIM = 256
_BLK = 1024
_COMMIT = 0.25


def _vq_tc_kernel(x_ref, e_ref, enc_ref, q_ref, loss_ref):
    i = pl.program_id(0)
    x = x_ref[...]
    e = e_ref[...]
    term1 = jnp.sum(x * x, axis=1, keepdims=True)
    term2 = jnp.sum(e * e, axis=1)
    term3 = jnp.dot(x, e.T, preferred_element_type=jnp.float32)
    dist = (term1 + term2[None, :]) - 2.0 * term3
    min_d = jnp.min(dist, axis=1, keepdims=True)
    col = jax.lax.broadcasted_iota(jnp.int32, (_BLK, _NUM_EMB), 1)
    idx = jnp.min(jnp.where(dist == min_d, col, _NUM_EMB), axis=1)
    enc = (col == idx[:, None]).astype(jnp.float32)
    enc_ref[...] = enc
    q = jnp.dot(enc, e, preferred_element_type=jnp.float32)
    q_ref[...] = q
    part = jnp.sum(min_d)

    @pl.when(i == 0)
    def _():
        loss_ref[0, 0] = 0.0

    loss_ref[0, 0] += part


def kernel(inputs, embedding):
    input_shape = inputs.shape
    flat = inputs.reshape(-1, _DIM)
    n = flat.shape[0]
    grid = (n // _BLK,)
    enc, q, loss_sum = pl.pallas_call(
        _vq_tc_kernel,
        grid=grid,
        in_specs=[
            pl.BlockSpec((_BLK, _DIM), lambda i: (i, 0)),
            pl.BlockSpec((_NUM_EMB, _DIM), lambda i: (0, 0)),
        ],
        out_specs=[
            pl.BlockSpec((_BLK, _NUM_EMB), lambda i: (i, 0)),
            pl.BlockSpec((_BLK, _DIM), lambda i: (i, 0)),
            pl.BlockSpec((1, 1), lambda i: (0, 0), memory_space=pltpu.SMEM),
        ],
        out_shape=[
            jax.ShapeDtypeStruct((n, _NUM_EMB), jnp.float32),
            jax.ShapeDtypeStruct((n, _DIM), jnp.float32),
            jax.ShapeDtypeStruct((1, 1), jnp.float32),
        ],
    )(flat, embedding)
    loss = loss_sum[0, 0] * ((1.0 + _COMMIT) / (n * _DIM))
    quantized = q.reshape(input_shape[0], -1)
    return (loss, quantized, enc)


# speedup vs baseline: 1.5019x; 1.0356x over previous
import jax
import jax.numpy as jnp
from jax.experimental import pallas as pl
from jax.experimental.pallas import tpu as pltpu

_NUM_EMB = 1024
_DIM = 256
_BLK = 2048
_COMMIT = 0.25


def _vq_tc_kernel(x_ref, e_ref, enc_ref, q_ref, loss_ref):
    i = pl.program_id(0)
    x = x_ref[...]
    e = e_ref[...]
    term1 = jnp.sum(x * x, axis=1, keepdims=True)
    term2 = jnp.sum(e * e, axis=1)
    term3 = jnp.dot(x, e.T, preferred_element_type=jnp.float32)
    dist = (term1 + term2[None, :]) - 2.0 * term3
    min_d = jnp.min(dist, axis=1, keepdims=True)
    col = jax.lax.broadcasted_iota(jnp.int32, (_BLK, _NUM_EMB), 1)
    idx = jnp.min(jnp.where(dist == min_d, col, _NUM_EMB), axis=1)
    enc = (col == idx[:, None]).astype(jnp.float32)
    enc_ref[...] = enc
    q = jnp.dot(enc, e, preferred_element_type=jnp.float32)
    q_ref[...] = q
    part = jnp.sum(min_d)

    @pl.when(i == 0)
    def _():
        loss_ref[0, 0] = 0.0

    loss_ref[0, 0] += part


def kernel(inputs, embedding):
    input_shape = inputs.shape
    flat = inputs.reshape(-1, _DIM)
    n = flat.shape[0]
    grid = (n // _BLK,)
    enc, q, loss_sum = pl.pallas_call(
        _vq_tc_kernel,
        grid=grid,
        in_specs=[
            pl.BlockSpec((_BLK, _DIM), lambda i: (i, 0)),
            pl.BlockSpec((_NUM_EMB, _DIM), lambda i: (0, 0)),
        ],
        out_specs=[
            pl.BlockSpec((_BLK, _NUM_EMB), lambda i: (i, 0)),
            pl.BlockSpec((_BLK, _DIM), lambda i: (i, 0)),
            pl.BlockSpec((1, 1), lambda i: (0, 0), memory_space=pltpu.SMEM),
        ],
        out_shape=[
            jax.ShapeDtypeStruct((n, _NUM_EMB), jnp.float32),
            jax.ShapeDtypeStruct((n, _DIM), jnp.float32),
            jax.ShapeDtypeStruct((1, 1), jnp.float32),
        ],
    )(flat, embedding)
    loss = loss_sum[0, 0] * ((1.0 + _COMMIT) / (n * _DIM))
    quantized = q.reshape(input_shape[0], -1)
    return (loss, quantized, enc)


# speedup vs baseline: 1.6929x; 1.1272x over previous
import jax
import jax.numpy as jnp
from jax.experimental import pallas as pl
from jax.experimental.pallas import tpu as pltpu

_NUM_EMB = 1024
_DIM = 256
_BLK = 2048
_COMMIT = 0.25


def _vq_tc_kernel(x_ref, e_ref, enc_ref, q_ref, loss_ref):
    i = pl.program_id(0)
    x = x_ref[...]
    e = e_ref[...]
    term1 = jnp.sum(x * x, axis=1, keepdims=True)
    term2 = jnp.sum(e * e, axis=1)
    term3 = jnp.dot(x, e.T, preferred_element_type=jnp.float32)
    dist = (term1 + term2[None, :]) - 2.0 * term3
    min_d = jnp.min(dist, axis=1, keepdims=True)
    col = jax.lax.broadcasted_iota(jnp.int32, (_BLK, _NUM_EMB), 1)
    idx = jnp.min(jnp.where(dist == min_d, col, _NUM_EMB), axis=1)
    enc = (col == idx[:, None]).astype(jnp.float32)
    enc_ref[...] = enc
    q = jnp.dot(enc, e, preferred_element_type=jnp.float32)
    nb = _BLK // 1024
    qr = q.reshape(nb, 1024 * _DIM)
    for s in range(8 // nb):
        @pl.when(i % (8 // nb) == s)
        def _(qr=qr, s=s):
            q_ref[s * nb:(s + 1) * nb, :] = qr
    part = jnp.sum(min_d)

    @pl.when(i == 0)
    def _():
        loss_ref[0, 0] = 0.0

    loss_ref[0, 0] += part


def kernel(inputs, embedding):
    input_shape = inputs.shape
    flat = inputs.reshape(-1, _DIM)
    n = flat.shape[0]
    grid = (n // _BLK,)
    enc, q, loss_sum = pl.pallas_call(
        _vq_tc_kernel,
        grid=grid,
        in_specs=[
            pl.BlockSpec((_BLK, _DIM), lambda i: (i, 0)),
            pl.BlockSpec((_NUM_EMB, _DIM), lambda i: (0, 0)),
        ],
        out_specs=[
            pl.BlockSpec((_BLK, _NUM_EMB), lambda i: (i, 0)),
            pl.BlockSpec((8, 1024 * _DIM), lambda i: (i // (8 // (_BLK // 1024)), 0)),
            pl.BlockSpec((1, 1), lambda i: (0, 0), memory_space=pltpu.SMEM),
        ],
        out_shape=[
            jax.ShapeDtypeStruct((n, _NUM_EMB), jnp.float32),
            jax.ShapeDtypeStruct((n // 1024, 1024 * _DIM), jnp.float32),
            jax.ShapeDtypeStruct((1, 1), jnp.float32),
        ],
    )(flat, embedding)
    loss = loss_sum[0, 0] * ((1.0 + _COMMIT) / (n * _DIM))
    return (loss, q, enc)


# speedup vs baseline: 1.8508x; 1.0932x over previous
import jax
import jax.numpy as jnp
from jax.experimental import pallas as pl
from jax.experimental.pallas import tpu as pltpu

_NUM_EMB = 1024
_DIM = 256
_NB = 8
_NP = 256
_BLK = _NB * _NP
_COMMIT = 0.25


def _vq_tc_kernel(x_ref, e_ref, enc_ref, q_ref, loss_ref):
    i = pl.program_id(0)
    x = x_ref[...].reshape(_BLK, _DIM)
    e = e_ref[...]
    term1 = jnp.sum(x * x, axis=1, keepdims=True)
    term2 = jnp.sum(e * e, axis=1)
    term3 = jnp.dot(x, e.T, preferred_element_type=jnp.float32)
    dist = (term1 + term2[None, :]) - 2.0 * term3
    min_d = jnp.min(dist, axis=1, keepdims=True)
    col = jax.lax.broadcasted_iota(jnp.int32, (_BLK, _NUM_EMB), 1)
    idx = jnp.min(jnp.where(dist == min_d, col, _NUM_EMB), axis=1)
    enc = (col == idx[:, None]).astype(jnp.float32)
    enc_ref[...] = enc.reshape(_NB, _NP, _NUM_EMB)
    q = jnp.dot(enc, e, preferred_element_type=jnp.float32)
    q_ref[...] = q.reshape(_NB, _NP * _DIM)
    part = jnp.sum(min_d)

    @pl.when(i == 0)
    def _():
        loss_ref[0, 0] = 0.0

    loss_ref[0, 0] += part


def kernel(inputs, embedding):
    b, t, d = inputs.shape
    n = b * t
    n_pc = t // _NP
    n_bg = b // _NB
    grid = (n_bg * n_pc,)
    enc3, q, loss_sum = pl.pallas_call(
        _vq_tc_kernel,
        grid=grid,
        in_specs=[
            pl.BlockSpec((_NB, _NP, _DIM), lambda i, n_pc=n_pc: (i // n_pc, i % n_pc, 0)),
            pl.BlockSpec((_NUM_EMB, _DIM), lambda i: (0, 0)),
        ],
        out_specs=[
            pl.BlockSpec((_NB, _NP, _NUM_EMB), lambda i, n_pc=n_pc: (i // n_pc, i % n_pc, 0)),
            pl.BlockSpec((_NB, _NP * _DIM), lambda i, n_pc=n_pc: (i // n_pc, i % n_pc)),
            pl.BlockSpec((1, 1), lambda i: (0, 0), memory_space=pltpu.SMEM),
        ],
        out_shape=[
            jax.ShapeDtypeStruct((b, t, _NUM_EMB), jnp.float32),
            jax.ShapeDtypeStruct((b, t * _DIM), jnp.float32),
            jax.ShapeDtypeStruct((1, 1), jnp.float32),
        ],
    )(inputs, embedding)
    loss = loss_sum[0, 0] * ((1.0 + _COMMIT) / (n * _DIM))
    enc = enc3.reshape(n, _NUM_EMB)
    return (loss, q, enc)
